# Initial kernel scaffold; baseline (speedup 1.0000x reference)
import jax
import jax.numpy as jnp
from jax.experimental import pallas as pl


def kernel(h, edge_index, e_f, W_l, W_e, W_q, W_k, W_v, gamma, beta):
    raise NotImplementedError("write your pallas kernel here")


# speedup vs baseline: 1.8164x; 1.8164x over previous
import functools

import jax
import jax.numpy as jnp
from jax import lax
from jax.experimental import pallas as pl
from jax.experimental.pallas import tpu as pltpu
from jax.experimental.pallas import tpu_sc as plsc

_N = 10000
_E = 160000
_D = 128
_DE = 11

_ROW = 304
_OFF_Z = 0
_OFF_U = 128
_OFF_P = 256
_OFF_Q = 272
_OFF_D = 288

_ACC_W = 144
_NC = 2
_NS = 16
_NTILES = _NC * _NS
_E_PAD = 163840
_EPT = _E_PAD // _NTILES
_B = 32
_NCHUNK = _EPT // _B
_N_PAD = 10240
_NPT = _N_PAD // _NS
_ZROWS = 8

_f32 = jnp.float32
_i32 = jnp.int32


def _dotT(a, b):
    return lax.dot_general(a, b, (((1,), (1,)), ((), ())), preferred_element_type=_f32)


def _dot(a, b):
    return lax.dot_general(a, b, (((1,), (0,)), ((), ())), preferred_element_type=_f32)


_BN = 400


def _node_prepass_body(h_ref, wl_ref, wq_ref, wk_ref, we_ref, out_ref):
    h = h_ref[...]
    wl = wl_ref[...]
    wq = wq_ref[...]
    wk = wk_ref[...]
    we = we_ref[...]
    z = _dotT(h, wl)
    u = _dot(_dotT(z, wq), wk)
    qpre = _dot(_dotT(z, wk), wq)
    p16 = _dot(u, we)
    q16 = _dot(qpre, we)
    d = jnp.sum(z * u, axis=1, keepdims=True)
    out_ref[:, 0:128] = z
    out_ref[:, 128:256] = u
    out_ref[:, 256:272] = p16
    out_ref[:, 272:288] = q16
    out_ref[:, 288:304] = jnp.concatenate(
        [d, jnp.zeros((_BN, 15), _f32)], axis=1)


_BE = 2048


def _edge_prepass_body(ef_ref, wq_ref, wk_ref, we_ref, out_ref):
    ef = ef_ref[...]
    we = we_ref[...]
    a1 = _dot(wq_ref[...], we)
    a2 = _dot(wk_ref[...], we)
    qe = lax.dot_general(a1, a2, (((0,), (0,)), ((), ())),
                         preferred_element_type=_f32)
    t = _dot(ef, qe)
    s22 = jnp.sum(t * ef, axis=1, keepdims=True)
    out_ref[...] = jnp.concatenate(
        [ef[:, :_DE], s22, jnp.zeros((_BE, 4), _f32)], axis=1)


def _sc_body(g_hbm, ed_hbm, src_hbm, dst_hbm, acc_hbm,
             src_v, dst_v, gs_v, gd_v, ed_v, out_v, zb_v, acc_sh, sem):
    cid = lax.axis_index("c")
    sid = lax.axis_index("s")
    tile = cid * _NS + sid
    iota = lax.iota(_i32, 16)

    for r in range(_ZROWS):
        for j in range(_ACC_W // 16):
            zb_v[r, pl.ds(j * 16, 16)] = jnp.zeros((16,), _f32)

    def _zero_chunk(k, carry):
        pltpu.sync_copy(zb_v, acc_sh.at[pl.ds(sid * _NPT + k * _ZROWS, _ZROWS)])
        return carry

    lax.fori_loop(0, _NPT // _ZROWS, _zero_chunk, 0)

    for r in range(_B):
        out_v[r, pl.ds(128, 16)] = jnp.zeros((16,), _f32)

    plsc.subcore_barrier()

    base_t = tile * _EPT

    def _chunk(c, carry):
        base = base_t + c * _B
        pltpu.sync_copy(src_hbm.at[pl.ds(base, _B)], src_v)
        pltpu.sync_copy(dst_hbm.at[pl.ds(base, _B)], dst_v)
        pltpu.sync_copy(ed_hbm.at[pl.ds(base, _B)], ed_v)
        pltpu.async_copy(g_hbm.at[src_v], gs_v, sem).wait()
        pltpu.async_copy(g_hbm.at[dst_v], gd_v, sem).wait()

        for g in range(_B // 16):
            rows = g * 16 + iota
            colZ = jnp.full((16,), _OFF_Z, _i32)
            colU = jnp.full((16,), _OFF_U, _i32)
            s00 = plsc.load_gather(gs_v, [rows, jnp.full((16,), _OFF_D, _i32)])
            s11 = plsc.load_gather(gd_v, [rows, jnp.full((16,), _OFF_D, _i32)])
            s22 = plsc.load_gather(ed_v, [rows, jnp.full((16,), _DE, _i32)])

            def _dot_step(f, carry2):
                s01a, s10a = carry2
                cz = colZ + f
                cu = colU + f
                us = plsc.load_gather(gs_v, [rows, cu])
                zs = plsc.load_gather(gs_v, [rows, cz])
                ud = plsc.load_gather(gd_v, [rows, cu])
                zd = plsc.load_gather(gd_v, [rows, cz])
                return (s01a + us * zd, s10a + ud * zs)

            s01, s10 = lax.fori_loop(
                0, _D, _dot_step,
                (jnp.zeros((16,), _f32), jnp.zeros((16,), _f32)))

            s02 = jnp.zeros((16,), _f32)
            s12 = jnp.zeros((16,), _f32)
            s20 = jnp.zeros((16,), _f32)
            s21 = jnp.zeros((16,), _f32)
            for f in range(_DE):
                ps = plsc.load_gather(gs_v, [rows, jnp.full((16,), _OFF_P + f, _i32)])
                pd = plsc.load_gather(gd_v, [rows, jnp.full((16,), _OFF_P + f, _i32)])
                qs = plsc.load_gather(gs_v, [rows, jnp.full((16,), _OFF_Q + f, _i32)])
                qd = plsc.load_gather(gd_v, [rows, jnp.full((16,), _OFF_Q + f, _i32)])
                ef = plsc.load_gather(ed_v, [rows, jnp.full((16,), f, _i32)])
                s02 = s02 + ps * ef
                s12 = s12 + pd * ef
                s20 = s20 + qs * ef
                s21 = s21 + qd * ef

            m0 = jnp.maximum(jnp.maximum(s00, s01), s02)
            e00 = jnp.exp(s00 - m0)
            e01 = jnp.exp(s01 - m0)
            e02 = jnp.exp(s02 - m0)
            i0 = 1.0 / (e00 + e01 + e02)
            m1 = jnp.maximum(jnp.maximum(s10, s11), s12)
            e10 = jnp.exp(s10 - m1)
            e11 = jnp.exp(s11 - m1)
            e12 = jnp.exp(s12 - m1)
            i1 = 1.0 / (e10 + e11 + e12)
            m2 = jnp.maximum(jnp.maximum(s20, s21), s22)
            e20 = jnp.exp(s20 - m2)
            e21 = jnp.exp(s21 - m2)
            e22 = jnp.exp(s22 - m2)
            i2 = 1.0 / (e20 + e21 + e22)
            w0 = e00 * i0 + e10 * i1 + e20 * i2
            w1 = e01 * i0 + e11 * i1 + e21 * i2
            w2 = e02 * i0 + e12 * i1 + e22 * i2

            eidx = base + g * 16 + iota
            valid = jnp.where(eidx < _E, 1.0, 0.0).astype(_f32)
            w0 = w0 * valid
            w1 = w1 * valid
            w2 = w2 * valid

            plsc.store_scatter(out_v, [rows, jnp.full((16,), 128, _i32)], w1)
            plsc.store_scatter(out_v, [rows, jnp.full((16,), 129, _i32)], valid)
            for f in range(_DE):
                ef = plsc.load_gather(ed_v, [rows, jnp.full((16,), f, _i32)])
                plsc.store_scatter(out_v, [rows, jnp.full((16,), 130 + f, _i32)],
                                   w2 * ef)

            def _vec_step(f, carry2):
                cz = colZ + f
                zs = plsc.load_gather(gs_v, [rows, cz])
                plsc.store_scatter(out_v, [rows, cz], w0 * zs)
                return carry2

            lax.fori_loop(0, _D, _vec_step, 0)

        pltpu.sync_copy(out_v, acc_sh.at[dst_v], add=True)
        return carry

    lax.fori_loop(0, _NCHUNK, _chunk, 0)

    plsc.subcore_barrier()
    lo = sid * _NPT
    pltpu.sync_copy(acc_sh.at[pl.ds(lo, _NPT)],
                    acc_hbm.at[cid, pl.ds(lo, _NPT)])


@functools.cache
def _sc_aggregate():
    return pl.kernel(
        _sc_body,
        out_type=jax.ShapeDtypeStruct((_NC, _N_PAD, _ACC_W), _f32),
        mesh=plsc.VectorSubcoreMesh(
            core_axis_name="c", subcore_axis_name="s",
            num_cores=_NC, num_subcores=_NS),
        compiler_params=pltpu.CompilerParams(use_tc_tiling_on_sc=False, needs_layout_passes=False),
        scratch_types=[
            pltpu.VMEM((_B,), _i32),
            pltpu.VMEM((_B,), _i32),
            pltpu.VMEM((_B, _ROW), _f32),
            pltpu.VMEM((_B, _ROW), _f32),
            pltpu.VMEM((_B, 16), _f32),
            pltpu.VMEM((_B, _ACC_W), _f32),
            pltpu.VMEM((_ZROWS, _ACC_W), _f32),
            pltpu.VMEM_SHARED((_N_PAD, _ACC_W), _f32),
            pltpu.SemaphoreType.DMA,
        ],
    )


def _post_body(acc_ref, z_ref, we_ref, wv_ref, g_ref, b_ref, out_ref):
    acc = acc_ref[0] + acc_ref[1]
    s0 = acc[:, 0:128]
    c1 = acc[:, 128:129]
    cnt = acc[:, 129:130]
    efa = acc[:, 130:141]
    z = z_ref[...]
    r2 = _dotT(efa, we_ref[...])
    pre = s0 + c1 * z + r2
    inv = 1.0 / jnp.maximum(cnt, 1.0)
    hn = _dotT(pre * inv, wv_ref[...])
    r = jnp.maximum(hn, 0.0)
    mean = jnp.mean(r, axis=1, keepdims=True)
    var = jnp.mean((r - mean) ** 2, axis=1, keepdims=True)
    out_ref[...] = (r - mean) * lax.rsqrt(var + 1e-5) * g_ref[...] + b_ref[...]


def kernel(h, edge_index, e_f, W_l, W_e, W_q, W_k, W_v, gamma, beta):
    we16 = jnp.pad(W_e, ((0, 0), (0, 16 - _DE)))

    G = pl.pallas_call(
        _node_prepass_body,
        grid=(_N // _BN,),
        in_specs=[
            pl.BlockSpec((_BN, _D), lambda i: (i, 0)),
            pl.BlockSpec((_D, _D), lambda i: (0, 0)),
            pl.BlockSpec((_D, _D), lambda i: (0, 0)),
            pl.BlockSpec((_D, _D), lambda i: (0, 0)),
            pl.BlockSpec((_D, 16), lambda i: (0, 0)),
        ],
        out_specs=pl.BlockSpec((_BN, _ROW), lambda i: (i, 0)),
        out_shape=jax.ShapeDtypeStruct((_N, _ROW), _f32),
    )(h, W_l, W_q, W_k, we16)

    ef16 = jnp.pad(e_f, ((0, _E_PAD - _E), (0, 16 - _DE)))
    ED = pl.pallas_call(
        _edge_prepass_body,
        grid=(_E_PAD // _BE,),
        in_specs=[
            pl.BlockSpec((_BE, 16), lambda i: (i, 0)),
            pl.BlockSpec((_D, _D), lambda i: (0, 0)),
            pl.BlockSpec((_D, _D), lambda i: (0, 0)),
            pl.BlockSpec((_D, 16), lambda i: (0, 0)),
        ],
        out_specs=pl.BlockSpec((_BE, 16), lambda i: (i, 0)),
        out_shape=jax.ShapeDtypeStruct((_E_PAD, 16), _f32),
    )(ef16, W_q, W_k, we16)

    srcp = jnp.pad(edge_index[0], (0, _E_PAD - _E))
    dstp = jnp.pad(edge_index[1], (0, _E_PAD - _E))

    acc = _sc_aggregate()(G, ED, srcp, dstp)

    out = pl.pallas_call(
        _post_body,
        grid=(_N // _BN,),
        in_specs=[
            pl.BlockSpec((_NC, _BN, _ACC_W), lambda i: (0, i, 0)),
            pl.BlockSpec((_BN, _D), lambda i: (i, 0)),
            pl.BlockSpec((_D, _DE), lambda i: (0, 0)),
            pl.BlockSpec((_D, _D), lambda i: (0, 0)),
            pl.BlockSpec((1, _D), lambda i: (0, 0)),
            pl.BlockSpec((1, _D), lambda i: (0, 0)),
        ],
        out_specs=pl.BlockSpec((_BN, _D), lambda i: (i, 0)),
        out_shape=jax.ShapeDtypeStruct((_N, _D), _f32),
    )(acc, G, W_e, W_v, gamma.reshape(1, _D), beta.reshape(1, _D))
    return out


# speedup vs baseline: 2.1099x; 1.1616x over previous
import functools

import jax
import jax.numpy as jnp
from jax import lax
from jax.experimental import pallas as pl
from jax.experimental.pallas import tpu as pltpu
from jax.experimental.pallas import tpu_sc as plsc

_N = 10000
_E = 160000
_D = 128
_DE = 11

_ROWF = 304
_ROW = 160
_OZ = 0
_OU = 64
_OP = 128
_OQ = 136
_OD = 144
_NZW = _D // 2

_EDW = 16
_ACC_W = 144
_NC = 2
_NS = 16
_NTILES = _NC * _NS
_E_PAD = 163840
_EPT = _E_PAD // _NTILES
_B = 32
_NCHUNK = _EPT // _B
_NPAIR = _NCHUNK // 2
_N_PAD = 10240
_NPT = _N_PAD // _NS
_ZROWS = 8

_f32 = jnp.float32
_i32 = jnp.int32
_bf16 = jnp.bfloat16


def _dotT(a, b):
    return lax.dot_general(a, b, (((1,), (1,)), ((), ())), preferred_element_type=_f32)


def _dot(a, b):
    return lax.dot_general(a, b, (((1,), (0,)), ((), ())), preferred_element_type=_f32)


_BN = 400


def _node_prepass_body(h_ref, wl_ref, wq_ref, wk_ref, we_ref, out_ref):
    h = h_ref[...]
    wl = wl_ref[...]
    wq = wq_ref[...]
    wk = wk_ref[...]
    we = we_ref[...]
    z = _dotT(h, wl)
    u = _dot(_dotT(z, wq), wk)
    qpre = _dot(_dotT(z, wk), wq)
    p16 = _dot(u, we)
    q16 = _dot(qpre, we)
    d = jnp.sum(z * u, axis=1, keepdims=True)
    out_ref[:, 0:128] = z
    out_ref[:, 128:256] = u
    out_ref[:, 256:272] = p16
    out_ref[:, 272:288] = q16
    out_ref[:, 288:304] = jnp.concatenate(
        [d, jnp.zeros((_BN, 15), _f32)], axis=1)


_BE = 2048


def _edge_prepass_body(ef_ref, wq_ref, wk_ref, we_ref, out_ref):
    ef = ef_ref[...]
    we = we_ref[...]
    a1 = _dot(wq_ref[...], we)
    a2 = _dot(wk_ref[...], we)
    qe = lax.dot_general(a1, a2, (((0,), (0,)), ((), ())),
                         preferred_element_type=_f32)
    t = _dot(ef, qe)
    s22 = jnp.sum(t * ef, axis=1, keepdims=True)
    out_ref[...] = jnp.concatenate(
        [ef[:, :_DE], jnp.zeros((_BE, 1), _f32), s22,
         jnp.zeros((_BE, 3), _f32)], axis=1)


def _pack_pairs(x):
    xb = x.astype(_bf16)
    xi = lax.bitcast_convert_type(xb, jnp.uint16).astype(jnp.uint32)
    pk = xi[:, 0::2] | (xi[:, 1::2] << 16)
    return lax.bitcast_convert_type(pk, _f32)


def _unp(w):
    return plsc.unpack(plsc.bitcast(w, _bf16), format=plsc.PackFormat.INTERLEAVED)


def _sc_body(g_hbm, ed_hbm, src_hbm, dst_hbm, acc_hbm,
             src_a, dst_a, gs_a, gd_a, ed_a,
             src_b, dst_b, gs_b, gd_b, ed_b,
             out_v, zb_v, acc_sh, sem_a, sem_b):
    cid = lax.axis_index("c")
    sid = lax.axis_index("s")
    tile = cid * _NS + sid
    iota = lax.iota(_i32, 16)
    zeros16 = jnp.zeros((16,), _f32)

    for r in range(_ZROWS):
        for j in range(_ACC_W // 16):
            zb_v[r, pl.ds(j * 16, 16)] = zeros16

    def _zero_chunk(k, carry):
        pltpu.sync_copy(zb_v, acc_sh.at[pl.ds(sid * _NPT + k * _ZROWS, _ZROWS)])
        return carry

    lax.fori_loop(0, _NPT // _ZROWS, _zero_chunk, 0)

    for r in range(_B):
        out_v[r, pl.ds(128, 16)] = zeros16

    plsc.subcore_barrier()

    base_t = tile * _EPT

    def _lin(c, sbuf, dbuf, ebuf):
        b = base_t + c * _B
        pltpu.sync_copy(src_hbm.at[pl.ds(b, _B)], sbuf)
        pltpu.sync_copy(dst_hbm.at[pl.ds(b, _B)], dbuf)
        pltpu.sync_copy(ed_hbm.at[pl.ds(b, _B)], ebuf)

    def _gat_issue(sbuf, dbuf, gs, gd, sem):
        pltpu.async_copy(g_hbm.at[sbuf], gs, sem)
        pltpu.async_copy(g_hbm.at[dbuf], gd, sem)

    def _gat_wait(sbuf, dbuf, gs, gd, sem):
        pltpu.make_async_copy(g_hbm.at[sbuf], gs, sem).wait()
        pltpu.make_async_copy(g_hbm.at[dbuf], gd, sem).wait()

    def _compute(gs, gd, ed, dbuf, c):
        base = base_t + c * _B
        for g in range(_B // 16):
            rows = g * 16 + iota

            def _dot_step(t, carry):
                s01a, s01b, s10a, s10b = carry
                bc4t = jnp.full((16,), 0, _i32) + 4 * t
                for k in range(4):
                    cz = bc4t + k
                    cu = cz + _OU
                    us = plsc.load_gather(gs, [rows, cu])
                    ud = plsc.load_gather(gd, [rows, cu])
                    zs = plsc.load_gather(gs, [rows, cz])
                    zd = plsc.load_gather(gd, [rows, cz])
                    us_l, us_h = _unp(us)
                    ud_l, ud_h = _unp(ud)
                    zs_l, zs_h = _unp(zs)
                    zd_l, zd_h = _unp(zd)
                    s01a = s01a + us_l * zd_l
                    s01b = s01b + us_h * zd_h
                    s10a = s10a + ud_l * zs_l
                    s10b = s10b + ud_h * zs_h
                return (s01a, s01b, s10a, s10b)

            s01a, s01b, s10a, s10b = lax.fori_loop(
                0, _NZW // 4, _dot_step, (zeros16, zeros16, zeros16, zeros16))
            s01 = s01a + s01b
            s10 = s10a + s10b

            s00 = plsc.load_gather(gs, [rows, jnp.full((16,), _OD, _i32)])
            s11 = plsc.load_gather(gd, [rows, jnp.full((16,), _OD, _i32)])
            s22 = plsc.load_gather(ed, [rows, jnp.full((16,), 12, _i32)])

            s02 = zeros16
            s12 = zeros16
            s20 = zeros16
            s21 = zeros16
            ef_l = [None] * 6
            ef_h = [None] * 6
            for w in range(6):
                cp = jnp.full((16,), _OP + w, _i32)
                cq = jnp.full((16,), _OQ + w, _i32)
                ps = plsc.load_gather(gs, [rows, cp])
                pd = plsc.load_gather(gd, [rows, cp])
                qs = plsc.load_gather(gs, [rows, cq])
                qd = plsc.load_gather(gd, [rows, cq])
                ps_l, ps_h = _unp(ps)
                pd_l, pd_h = _unp(pd)
                qs_l, qs_h = _unp(qs)
                qd_l, qd_h = _unp(qd)
                ef_l[w] = plsc.load_gather(ed, [rows, jnp.full((16,), 2 * w, _i32)])
                ef_h[w] = plsc.load_gather(ed, [rows, jnp.full((16,), 2 * w + 1, _i32)])
                s02 = s02 + ps_l * ef_l[w] + ps_h * ef_h[w]
                s12 = s12 + pd_l * ef_l[w] + pd_h * ef_h[w]
                s20 = s20 + qs_l * ef_l[w] + qs_h * ef_h[w]
                s21 = s21 + qd_l * ef_l[w] + qd_h * ef_h[w]

            m0 = jnp.maximum(jnp.maximum(s00, s01), s02)
            e00 = jnp.exp(s00 - m0)
            e01 = jnp.exp(s01 - m0)
            e02 = jnp.exp(s02 - m0)
            i0 = 1.0 / (e00 + e01 + e02)
            m1 = jnp.maximum(jnp.maximum(s10, s11), s12)
            e10 = jnp.exp(s10 - m1)
            e11 = jnp.exp(s11 - m1)
            e12 = jnp.exp(s12 - m1)
            i1 = 1.0 / (e10 + e11 + e12)
            m2 = jnp.maximum(jnp.maximum(s20, s21), s22)
            e20 = jnp.exp(s20 - m2)
            e21 = jnp.exp(s21 - m2)
            e22 = jnp.exp(s22 - m2)
            i2 = 1.0 / (e20 + e21 + e22)
            w0 = e00 * i0 + e10 * i1 + e20 * i2
            w1 = e01 * i0 + e11 * i1 + e21 * i2
            w2 = e02 * i0 + e12 * i1 + e22 * i2

            eidx = base + g * 16 + iota
            valid = jnp.where(eidx < _E, 1.0, 0.0).astype(_f32)
            w0 = w0 * valid
            w1 = w1 * valid
            w2 = w2 * valid

            plsc.store_scatter(out_v, [rows, jnp.full((16,), 128, _i32)], w1)
            plsc.store_scatter(out_v, [rows, jnp.full((16,), 129, _i32)], valid)
            for w in range(6):
                plsc.store_scatter(
                    out_v, [rows, jnp.full((16,), 130 + 2 * w, _i32)], w2 * ef_l[w])
                plsc.store_scatter(
                    out_v, [rows, jnp.full((16,), 131 + 2 * w, _i32)], w2 * ef_h[w])

            def _vec_step(t, carry):
                bc4t = jnp.full((16,), 0, _i32) + 4 * t
                for k in range(4):
                    cz = bc4t + k
                    zs = plsc.load_gather(gs, [rows, cz])
                    zs_l, zs_h = _unp(zs)
                    ce = cz + cz
                    plsc.store_scatter(out_v, [rows, ce], w0 * zs_l)
                    plsc.store_scatter(out_v, [rows, ce + 1], w0 * zs_h)
                return carry

            lax.fori_loop(0, _NZW // 4, _vec_step, 0)

        pltpu.sync_copy(out_v, acc_sh.at[dbuf], add=True)

    _lin(0, src_a, dst_a, ed_a)
    _gat_issue(src_a, dst_a, gs_a, gd_a, sem_a)

    def _pair(c2, carry):
        c0 = 2 * c2
        _lin(c0 + 1, src_b, dst_b, ed_b)
        _gat_issue(src_b, dst_b, gs_b, gd_b, sem_b)
        _gat_wait(src_a, dst_a, gs_a, gd_a, sem_a)
        _compute(gs_a, gd_a, ed_a, dst_a, c0)

        @pl.when(c2 + 1 < _NPAIR)
        def _():
            _lin(c0 + 2, src_a, dst_a, ed_a)
            _gat_issue(src_a, dst_a, gs_a, gd_a, sem_a)

        _gat_wait(src_b, dst_b, gs_b, gd_b, sem_b)
        _compute(gs_b, gd_b, ed_b, dst_b, c0 + 1)
        return carry

    lax.fori_loop(0, _NPAIR, _pair, 0)

    plsc.subcore_barrier()
    lo = sid * _NPT
    pltpu.sync_copy(acc_sh.at[pl.ds(lo, _NPT)],
                    acc_hbm.at[cid, pl.ds(lo, _NPT)])


@functools.cache
def _sc_aggregate():
    return pl.kernel(
        _sc_body,
        out_type=jax.ShapeDtypeStruct((_NC, _N_PAD, _ACC_W), _f32),
        mesh=plsc.VectorSubcoreMesh(
            core_axis_name="c", subcore_axis_name="s",
            num_cores=_NC, num_subcores=_NS),
        compiler_params=pltpu.CompilerParams(
            use_tc_tiling_on_sc=False, needs_layout_passes=False),
        scratch_types=[
            pltpu.VMEM((_B,), _i32),
            pltpu.VMEM((_B,), _i32),
            pltpu.VMEM((_B, _ROW), _f32),
            pltpu.VMEM((_B, _ROW), _f32),
            pltpu.VMEM((_B, _EDW), _f32),
            pltpu.VMEM((_B,), _i32),
            pltpu.VMEM((_B,), _i32),
            pltpu.VMEM((_B, _ROW), _f32),
            pltpu.VMEM((_B, _ROW), _f32),
            pltpu.VMEM((_B, _EDW), _f32),
            pltpu.VMEM((_B, _ACC_W), _f32),
            pltpu.VMEM((_ZROWS, _ACC_W), _f32),
            pltpu.VMEM_SHARED((_N_PAD, _ACC_W), _f32),
            pltpu.SemaphoreType.DMA,
            pltpu.SemaphoreType.DMA,
        ],
    )


def _post_body(acc_ref, z_ref, we_ref, wv_ref, g_ref, b_ref, out_ref):
    acc = acc_ref[0] + acc_ref[1]
    s0 = acc[:, 0:128]
    c1 = acc[:, 128:129]
    cnt = acc[:, 129:130]
    efa = acc[:, 130:141]
    z = z_ref[...]
    r2 = _dotT(efa, we_ref[...])
    pre = s0 + c1 * z + r2
    inv = 1.0 / jnp.maximum(cnt, 1.0)
    hn = _dotT(pre * inv, wv_ref[...])
    r = jnp.maximum(hn, 0.0)
    mean = jnp.mean(r, axis=1, keepdims=True)
    var = jnp.mean((r - mean) ** 2, axis=1, keepdims=True)
    out_ref[...] = (r - mean) * lax.rsqrt(var + 1e-5) * g_ref[...] + b_ref[...]


def kernel(h, edge_index, e_f, W_l, W_e, W_q, W_k, W_v, gamma, beta):
    we16 = jnp.pad(W_e, ((0, 0), (0, 16 - _DE)))

    G = pl.pallas_call(
        _node_prepass_body,
        grid=(_N // _BN,),
        in_specs=[
            pl.BlockSpec((_BN, _D), lambda i: (i, 0)),
            pl.BlockSpec((_D, _D), lambda i: (0, 0)),
            pl.BlockSpec((_D, _D), lambda i: (0, 0)),
            pl.BlockSpec((_D, _D), lambda i: (0, 0)),
            pl.BlockSpec((_D, 16), lambda i: (0, 0)),
        ],
        out_specs=pl.BlockSpec((_BN, _ROWF), lambda i: (i, 0)),
        out_shape=jax.ShapeDtypeStruct((_N, _ROWF), _f32),
    )(h, W_l, W_q, W_k, we16)

    Gp = jnp.concatenate([
        _pack_pairs(G[:, 0:128]),
        _pack_pairs(G[:, 128:256]),
        _pack_pairs(G[:, 256:272]),
        _pack_pairs(G[:, 272:288]),
        G[:, 288:289],
        jnp.zeros((_N, _ROW - 145), _f32),
    ], axis=1)

    ef16 = jnp.pad(e_f, ((0, _E_PAD - _E), (0, 16 - _DE)))
    ED = pl.pallas_call(
        _edge_prepass_body,
        grid=(_E_PAD // _BE,),
        in_specs=[
            pl.BlockSpec((_BE, 16), lambda i: (i, 0)),
            pl.BlockSpec((_D, _D), lambda i: (0, 0)),
            pl.BlockSpec((_D, _D), lambda i: (0, 0)),
            pl.BlockSpec((_D, 16), lambda i: (0, 0)),
        ],
        out_specs=pl.BlockSpec((_BE, 16), lambda i: (i, 0)),
        out_shape=jax.ShapeDtypeStruct((_E_PAD, 16), _f32),
    )(ef16, W_q, W_k, we16)

    srcp = jnp.pad(edge_index[0], (0, _E_PAD - _E))
    dstp = jnp.pad(edge_index[1], (0, _E_PAD - _E))

    acc = _sc_aggregate()(Gp, ED, srcp, dstp)

    out = pl.pallas_call(
        _post_body,
        grid=(_N // _BN,),
        in_specs=[
            pl.BlockSpec((_NC, _BN, _ACC_W), lambda i: (0, i, 0)),
            pl.BlockSpec((_BN, _D), lambda i: (i, 0)),
            pl.BlockSpec((_D, _DE), lambda i: (0, 0)),
            pl.BlockSpec((_D, _D), lambda i: (0, 0)),
            pl.BlockSpec((1, _D), lambda i: (0, 0)),
            pl.BlockSpec((1, _D), lambda i: (0, 0)),
        ],
        out_specs=pl.BlockSpec((_BN, _D), lambda i: (i, 0)),
        out_shape=jax.ShapeDtypeStruct((_N, _D), _f32),
    )(acc, G, W_e, W_v, gamma.reshape(1, _D), beta.reshape(1, _D))
    return out
